# baseline (device time: 440037 ns/iter reference)
import jax
import jax.numpy as jnp
from jax import lax
from jax.experimental import pallas as pl
from jax.experimental.pallas import tpu as pltpu

X_SIZE = 2
CS = 128
UNROLL = 4


def kernel(O, Wo):
    B, S, Hs, D = O.shape
    K = Hs * D
    N = Wo.shape[1]
    s_half = S // X_SIZE
    per_b = s_half // CS
    nc = B * per_b
    assert nc % UNROLL == 0 and nc >= 2 * UNROLL

    O3 = O.reshape(B, S, K)

    def body(o_ref, wo_ref, out_ref, ar_buf, al_buf, send_buf, acc_buf, rtmp,
             ar_sems, al_sems, send_sems, recv_sems, rtmp_sems, wb_sems):
        my_x = lax.axis_index("x")
        my_y = lax.axis_index("y")
        my_z = lax.axis_index("z")
        partner = (1 - my_x, my_y, my_z)

        my_s0 = my_x * s_half
        their_s0 = (1 - my_x) * s_half

        def chunk_bso(c):
            return c // per_b, (c % per_b) * CS

        def out_chunk(c):
            b, so = chunk_bso(c)
            return out_ref.at[b, pl.ds(so, CS), :]

        def a_copy(c, buf, sems, slot, s0):
            b, so = chunk_bso(c)
            return pltpu.make_async_copy(
                o_ref.at[b, pl.ds(s0 + so, CS), :],
                buf.at[slot], sems.at[slot])

        def send_rdma(c, slot):
            return pltpu.make_async_remote_copy(
                src_ref=send_buf.at[slot],
                dst_ref=out_chunk(c),
                send_sem=send_sems.at[slot],
                recv_sem=recv_sems.at[c],
                device_id=partner,
                device_id_type=pl.DeviceIdType.MESH)

        def wb_copy(c, acc_slot, sem_slot):
            return pltpu.make_async_copy(
                acc_buf.at[acc_slot], out_chunk(c), wb_sems.at[sem_slot])

        def add_received(c, acc_slot, rtmp_slot):
            send_rdma(c, 0).wait_recv()
            rcp = pltpu.make_async_copy(
                out_chunk(c), rtmp.at[rtmp_slot], rtmp_sems.at[rtmp_slot])
            rcp.start()
            rcp.wait()
            acc_buf[acc_slot, :, :] = acc_buf[acc_slot] + rtmp[rtmp_slot]
            wb_copy(c, acc_slot, rtmp_slot).start()

        barrier = pltpu.get_barrier_semaphore()
        pl.semaphore_signal(barrier, inc=1, device_id=partner,
                            device_id_type=pl.DeviceIdType.MESH)
        pl.semaphore_wait(barrier, 1)

        a_copy(0, ar_buf, ar_sems, 0, their_s0).start()
        a_copy(1, ar_buf, ar_sems, 1, their_s0).start()
        a_copy(0, al_buf, al_sems, 0, my_s0).start()
        a_copy(1, al_buf, al_sems, 1, my_s0).start()

        def step(p, k):
            c = UNROLL * p + k
            ab = k % 2

            a_copy(c, ar_buf, ar_sems, ab, their_s0).wait()

            @pl.when(c >= UNROLL)
            def _():
                send_rdma(c - UNROLL, k).wait_send()

            send_buf[k, :, :] = jnp.dot(
                ar_buf[ab], wo_ref[:, :],
                preferred_element_type=jnp.float32)

            @pl.when(c + 2 < nc)
            def _():
                a_copy(c + 2, ar_buf, ar_sems, ab, their_s0).start()

            send_rdma(c, k).start()

            a_copy(c, al_buf, al_sems, ab, my_s0).wait()

            @pl.when(c >= UNROLL)
            def _():
                wb_copy(c - UNROLL, k, k % 2).wait()

            acc_buf[k, :, :] = jnp.dot(
                al_buf[ab], wo_ref[:, :],
                preferred_element_type=jnp.float32)

            @pl.when(c + 2 < nc)
            def _():
                a_copy(c + 2, al_buf, al_sems, ab, my_s0).start()

            @pl.when(c >= 2)
            def _():
                add_received(c - 2, (k - 2) % UNROLL, k % 2)

        def quad(p, carry):
            for k in range(UNROLL):
                step(p, k)
            return carry

        lax.fori_loop(0, nc // UNROLL, quad, 0)

        wb_copy(nc - UNROLL, (nc - UNROLL) % UNROLL, nc % 2).wait()
        add_received(nc - 2, (nc - 2) % UNROLL, nc % 2)
        wb_copy(nc - UNROLL + 1, (nc - UNROLL + 1) % UNROLL, (nc + 1) % 2).wait()
        add_received(nc - 1, (nc - 1) % UNROLL, (nc + 1) % 2)
        for i in range(UNROLL):
            send_rdma(nc - UNROLL + i, i).wait_send()
        wb_copy(nc - 2, (nc - 2) % UNROLL, nc % 2).wait()
        wb_copy(nc - 1, (nc - 1) % UNROLL, (nc + 1) % 2).wait()

    return pl.pallas_call(
        body,
        out_shape=jax.ShapeDtypeStruct((B, s_half, N), jnp.float32),
        in_specs=[pl.BlockSpec(memory_space=pl.ANY),
                  pl.BlockSpec(memory_space=pltpu.VMEM)],
        out_specs=pl.BlockSpec(memory_space=pl.ANY),
        scratch_shapes=[
            pltpu.VMEM((2, CS, K), jnp.float32),
            pltpu.VMEM((2, CS, K), jnp.float32),
            pltpu.VMEM((UNROLL, CS, N), jnp.float32),
            pltpu.VMEM((UNROLL, CS, N), jnp.float32),
            pltpu.VMEM((2, CS, N), jnp.float32),
            pltpu.SemaphoreType.DMA((2,)),
            pltpu.SemaphoreType.DMA((2,)),
            pltpu.SemaphoreType.DMA((UNROLL,)),
            pltpu.SemaphoreType.DMA((nc,)),
            pltpu.SemaphoreType.DMA((2,)),
            pltpu.SemaphoreType.DMA((2,)),
        ],
        compiler_params=pltpu.CompilerParams(
            collective_id=0,
            vmem_limit_bytes=61 * 1024 * 1024),
    )(O3, Wo)


# device time: 439924 ns/iter; 1.0003x vs baseline; 1.0003x over previous
import jax
import jax.numpy as jnp
from jax import lax
from jax.experimental import pallas as pl
from jax.experimental.pallas import tpu as pltpu

X_SIZE = 2
CS = 128
UNROLL = 4


def kernel(O, Wo):
    B, S, Hs, D = O.shape
    K = Hs * D
    N = Wo.shape[1]
    s_half = S // X_SIZE
    per_b = s_half // CS
    nc = B * per_b
    assert nc % UNROLL == 0 and nc >= 2 * UNROLL

    O3 = O.reshape(B, S, K)

    def body(o_ref, wo_ref, out_ref, ar_buf, al_buf, send_buf, acc_buf, rtmp,
             ar_sems, al_sems, send_sems, recv_sems, rtmp_sems, wb_sems):
        my_x = lax.axis_index("x")
        my_y = lax.axis_index("y")
        my_z = lax.axis_index("z")
        partner = (1 - my_x, my_y, my_z)

        my_s0 = my_x * s_half
        their_s0 = (1 - my_x) * s_half

        def chunk_bso(c):
            return c // per_b, (c % per_b) * CS

        def out_chunk(c):
            b, so = chunk_bso(c)
            return out_ref.at[b, pl.ds(so, CS), :]

        def a_copy(c, buf, sems, slot, s0):
            b, so = chunk_bso(c)
            return pltpu.make_async_copy(
                o_ref.at[b, pl.ds(s0 + so, CS), :],
                buf.at[slot], sems.at[slot])

        def send_rdma(c, slot):
            return pltpu.make_async_remote_copy(
                src_ref=send_buf.at[slot],
                dst_ref=out_chunk(c),
                send_sem=send_sems.at[slot],
                recv_sem=recv_sems.at[c],
                device_id=partner,
                device_id_type=pl.DeviceIdType.MESH)

        def wb_copy(c, acc_slot, sem_slot):
            return pltpu.make_async_copy(
                acc_buf.at[acc_slot], out_chunk(c), wb_sems.at[sem_slot])

        def rcp_copy(c, rtmp_slot):
            return pltpu.make_async_copy(
                out_chunk(c), rtmp.at[rtmp_slot], rtmp_sems.at[rtmp_slot])

        def start_readback(c, rtmp_slot):
            send_rdma(c, 0).wait_recv()
            rcp_copy(c, rtmp_slot).start()

        def add_received(c, acc_slot, rtmp_slot):
            rcp_copy(c, rtmp_slot).wait()
            acc_buf[acc_slot, :, :] = acc_buf[acc_slot] + rtmp[rtmp_slot]
            wb_copy(c, acc_slot, rtmp_slot).start()

        barrier = pltpu.get_barrier_semaphore()
        pl.semaphore_signal(barrier, inc=1, device_id=partner,
                            device_id_type=pl.DeviceIdType.MESH)
        pl.semaphore_wait(barrier, 1)

        a_copy(0, ar_buf, ar_sems, 0, their_s0).start()
        a_copy(1, ar_buf, ar_sems, 1, their_s0).start()
        a_copy(0, al_buf, al_sems, 0, my_s0).start()
        a_copy(1, al_buf, al_sems, 1, my_s0).start()

        def step(p, k):
            c = UNROLL * p + k
            ab = k % 2

            a_copy(c, ar_buf, ar_sems, ab, their_s0).wait()

            @pl.when(c >= UNROLL)
            def _():
                send_rdma(c - UNROLL, k).wait_send()

            send_buf[k, :, :] = jnp.dot(
                ar_buf[ab], wo_ref[:, :],
                preferred_element_type=jnp.float32)

            @pl.when(c + 2 < nc)
            def _():
                a_copy(c + 2, ar_buf, ar_sems, ab, their_s0).start()

            send_rdma(c, k).start()

            a_copy(c, al_buf, al_sems, ab, my_s0).wait()

            @pl.when(c >= UNROLL)
            def _():
                wb_copy(c - UNROLL, k, k % 2).wait()

            acc_buf[k, :, :] = jnp.dot(
                al_buf[ab], wo_ref[:, :],
                preferred_element_type=jnp.float32)

            @pl.when(c + 2 < nc)
            def _():
                a_copy(c + 2, al_buf, al_sems, ab, my_s0).start()

            @pl.when(c >= 2)
            def _():
                start_readback(c - 2, k % 2)

            @pl.when(c >= 3)
            def _():
                add_received(c - 3, (k - 3) % UNROLL, (k - 3) % 2)

        def quad(p, carry):
            for k in range(UNROLL):
                step(p, k)
            return carry

        lax.fori_loop(0, nc // UNROLL, quad, 0)

        add_received(nc - 3, (nc - 3) % UNROLL, (nc - 3) % 2)
        start_readback(nc - 2, (nc - 2) % 2)
        wb_copy(nc - UNROLL, (nc - UNROLL) % UNROLL, nc % 2).wait()
        add_received(nc - 2, (nc - 2) % UNROLL, (nc - 2) % 2)
        start_readback(nc - 1, (nc - 1) % 2)
        wb_copy(nc - 3, (nc - 3) % UNROLL, (nc - 3) % 2).wait()
        add_received(nc - 1, (nc - 1) % UNROLL, (nc - 1) % 2)
        for i in range(UNROLL):
            send_rdma(nc - UNROLL + i, i).wait_send()
        wb_copy(nc - 2, (nc - 2) % UNROLL, nc % 2).wait()
        wb_copy(nc - 1, (nc - 1) % UNROLL, (nc + 1) % 2).wait()

    return pl.pallas_call(
        body,
        out_shape=jax.ShapeDtypeStruct((B, s_half, N), jnp.float32),
        in_specs=[pl.BlockSpec(memory_space=pl.ANY),
                  pl.BlockSpec(memory_space=pltpu.VMEM)],
        out_specs=pl.BlockSpec(memory_space=pl.ANY),
        scratch_shapes=[
            pltpu.VMEM((2, CS, K), jnp.float32),
            pltpu.VMEM((2, CS, K), jnp.float32),
            pltpu.VMEM((UNROLL, CS, N), jnp.float32),
            pltpu.VMEM((UNROLL, CS, N), jnp.float32),
            pltpu.VMEM((2, CS, N), jnp.float32),
            pltpu.SemaphoreType.DMA((2,)),
            pltpu.SemaphoreType.DMA((2,)),
            pltpu.SemaphoreType.DMA((UNROLL,)),
            pltpu.SemaphoreType.DMA((nc,)),
            pltpu.SemaphoreType.DMA((2,)),
            pltpu.SemaphoreType.DMA((2,)),
        ],
        compiler_params=pltpu.CompilerParams(
            collective_id=0,
            vmem_limit_bytes=61 * 1024 * 1024),
    )(O3, Wo)


# device time: 404378 ns/iter; 1.0882x vs baseline; 1.0879x over previous
import jax
import jax.numpy as jnp
from jax import lax
from jax.experimental import pallas as pl
from jax.experimental.pallas import tpu as pltpu

X_SIZE = 2
CS = 128
UNROLL = 4


def kernel(O, Wo):
    B, S, Hs, D = O.shape
    K = Hs * D
    N = Wo.shape[1]
    s_half = S // X_SIZE
    per_b = s_half // CS
    nc = B * per_b
    assert nc % UNROLL == 0 and nc >= 2 * UNROLL


    def body(o_ref, wo_ref, out_ref, ar_buf, al_buf, send_buf, acc_buf, rtmp,
             ar_sems, al_sems, send_sems, recv_sems, rtmp_sems, wb_sems):
        my_x = lax.axis_index("x")
        my_y = lax.axis_index("y")
        my_z = lax.axis_index("z")
        partner = (1 - my_x, my_y, my_z)

        my_s0 = my_x * s_half
        their_s0 = (1 - my_x) * s_half

        def chunk_bso(c):
            return c // per_b, (c % per_b) * CS

        def out_chunk(c):
            b, so = chunk_bso(c)
            return out_ref.at[b, pl.ds(so, CS), :]

        def a_copy(c, buf, sems, slot, s0):
            b, so = chunk_bso(c)
            return pltpu.make_async_copy(
                o_ref.at[b, pl.ds(s0 + so, CS), :, :],
                buf.at[slot], sems.at[slot])

        def send_rdma(c, slot):
            return pltpu.make_async_remote_copy(
                src_ref=send_buf.at[slot],
                dst_ref=out_chunk(c),
                send_sem=send_sems.at[slot],
                recv_sem=recv_sems.at[c],
                device_id=partner,
                device_id_type=pl.DeviceIdType.MESH)

        def wb_copy(c, acc_slot, sem_slot):
            return pltpu.make_async_copy(
                acc_buf.at[acc_slot], out_chunk(c), wb_sems.at[sem_slot])

        def rcp_copy(c, rtmp_slot):
            return pltpu.make_async_copy(
                out_chunk(c), rtmp.at[rtmp_slot], rtmp_sems.at[rtmp_slot])

        def start_readback(c, rtmp_slot):
            send_rdma(c, 0).wait_recv()
            rcp_copy(c, rtmp_slot).start()

        def add_received(c, acc_slot, rtmp_slot):
            rcp_copy(c, rtmp_slot).wait()
            acc_buf[acc_slot, :, :] = acc_buf[acc_slot] + rtmp[rtmp_slot]
            wb_copy(c, acc_slot, rtmp_slot).start()

        barrier = pltpu.get_barrier_semaphore()
        pl.semaphore_signal(barrier, inc=1, device_id=partner,
                            device_id_type=pl.DeviceIdType.MESH)
        pl.semaphore_wait(barrier, 1)

        a_copy(0, ar_buf, ar_sems, 0, their_s0).start()
        a_copy(1, ar_buf, ar_sems, 1, their_s0).start()
        a_copy(0, al_buf, al_sems, 0, my_s0).start()
        a_copy(1, al_buf, al_sems, 1, my_s0).start()

        def step(p, k):
            c = UNROLL * p + k
            ab = k % 2

            a_copy(c, ar_buf, ar_sems, ab, their_s0).wait()

            @pl.when(c >= UNROLL)
            def _():
                send_rdma(c - UNROLL, k).wait_send()

            send_buf[k, :, :] = jnp.dot(
                ar_buf[ab].reshape(CS, K), wo_ref[:, :],
                preferred_element_type=jnp.float32)

            @pl.when(c + 2 < nc)
            def _():
                a_copy(c + 2, ar_buf, ar_sems, ab, their_s0).start()

            send_rdma(c, k).start()

            a_copy(c, al_buf, al_sems, ab, my_s0).wait()

            @pl.when(c >= UNROLL)
            def _():
                wb_copy(c - UNROLL, k, k % 2).wait()

            acc_buf[k, :, :] = jnp.dot(
                al_buf[ab].reshape(CS, K), wo_ref[:, :],
                preferred_element_type=jnp.float32)

            @pl.when(c + 2 < nc)
            def _():
                a_copy(c + 2, al_buf, al_sems, ab, my_s0).start()

            @pl.when(c >= 2)
            def _():
                start_readback(c - 2, k % 2)

            @pl.when(c >= 3)
            def _():
                add_received(c - 3, (k - 3) % UNROLL, (k - 3) % 2)

        def quad(p, carry):
            for k in range(UNROLL):
                step(p, k)
            return carry

        lax.fori_loop(0, nc // UNROLL, quad, 0)

        add_received(nc - 3, (nc - 3) % UNROLL, (nc - 3) % 2)
        start_readback(nc - 2, (nc - 2) % 2)
        wb_copy(nc - UNROLL, (nc - UNROLL) % UNROLL, nc % 2).wait()
        add_received(nc - 2, (nc - 2) % UNROLL, (nc - 2) % 2)
        start_readback(nc - 1, (nc - 1) % 2)
        wb_copy(nc - 3, (nc - 3) % UNROLL, (nc - 3) % 2).wait()
        add_received(nc - 1, (nc - 1) % UNROLL, (nc - 1) % 2)
        for i in range(UNROLL):
            send_rdma(nc - UNROLL + i, i).wait_send()
        wb_copy(nc - 2, (nc - 2) % UNROLL, nc % 2).wait()
        wb_copy(nc - 1, (nc - 1) % UNROLL, (nc + 1) % 2).wait()

    return pl.pallas_call(
        body,
        out_shape=jax.ShapeDtypeStruct((B, s_half, N), jnp.float32),
        in_specs=[pl.BlockSpec(memory_space=pl.ANY),
                  pl.BlockSpec(memory_space=pltpu.VMEM)],
        out_specs=pl.BlockSpec(memory_space=pl.ANY),
        scratch_shapes=[
            pltpu.VMEM((2, CS, Hs, D), jnp.float32),
            pltpu.VMEM((2, CS, Hs, D), jnp.float32),
            pltpu.VMEM((UNROLL, CS, N), jnp.float32),
            pltpu.VMEM((UNROLL, CS, N), jnp.float32),
            pltpu.VMEM((2, CS, N), jnp.float32),
            pltpu.SemaphoreType.DMA((2,)),
            pltpu.SemaphoreType.DMA((2,)),
            pltpu.SemaphoreType.DMA((UNROLL,)),
            pltpu.SemaphoreType.DMA((nc,)),
            pltpu.SemaphoreType.DMA((2,)),
            pltpu.SemaphoreType.DMA((2,)),
        ],
        compiler_params=pltpu.CompilerParams(
            collective_id=0,
            vmem_limit_bytes=61 * 1024 * 1024),
    )(O, Wo)
